# baseline (speedup 1.0000x reference)
import jax
import jax.numpy as jnp
from jax import lax
from jax.experimental import pallas as pl
from jax.experimental.pallas import tpu as pltpu
from jax.experimental.pallas import tpu_sc as plsc

N = 10000
E = 320000
D_IN = 128
HID = 64
D_OUT = 64
HALF = 32
ALPHA = 0.1
K = 10

NC = 2
NS = 16
NP = 10240
ROWS_PT = NP // NS

E_PAD = 327680
SCAN_E = 512
NSCAN = E_PAD // SCAN_E

CHUNK = 128
NBUF = 4
CAP = 22400
CAPG = CAP - 512

_BN = 2000


def _mlp_body(x_ref, w1_ref, b1_ref, w2_ref, b2_ref, o_ref, oa_ref):
    h = jnp.dot(x_ref[...], w1_ref[...], preferred_element_type=jnp.float32)
    h = jnp.maximum(h + b1_ref[...], 0.0)
    o = jnp.dot(h, w2_ref[...], preferred_element_type=jnp.float32)
    o = o + b2_ref[...]
    o_ref[...] = o
    oa_ref[...] = o * ALPHA


def _mlp(x, W1, b1, W2, b2):
    return pl.pallas_call(
        _mlp_body,
        grid=(N // _BN,),
        in_specs=[
            pl.BlockSpec((_BN, D_IN), lambda i: (i, 0)),
            pl.BlockSpec((D_IN, HID), lambda i: (0, 0)),
            pl.BlockSpec((1, HID), lambda i: (0, 0)),
            pl.BlockSpec((HID, D_OUT), lambda i: (0, 0)),
            pl.BlockSpec((1, D_OUT), lambda i: (0, 0)),
        ],
        out_specs=[pl.BlockSpec((_BN, D_OUT), lambda i: (i, 0)),
                   pl.BlockSpec((_BN, D_OUT), lambda i: (i, 0))],
        out_shape=[jax.ShapeDtypeStruct((N, D_OUT), jnp.float32),
                   jax.ShapeDtypeStruct((N, D_OUT), jnp.float32)],
    )(x, W1, b1.reshape(1, HID), W2, b2.reshape(1, D_OUT))


def _propagate_body(h_hbm, h01_hbm, src_hbm, dst_hbm, val_hbm, out_hbm,
                    bsrc, bdst, bval, acc, bufs, st_src, st_dst, st_val,
                    zsp, gsems, ssems):
    c = lax.axis_index("c")
    s = lax.axis_index("s")
    row0 = s * ROWS_PT
    iota = lax.iota(jnp.int32, 16)

    def fire_scan(b, k):
        pltpu.async_copy(src_hbm.at[pl.ds(k * SCAN_E, SCAN_E)],
                         st_src.at[b], ssems.at[b])
        pltpu.async_copy(dst_hbm.at[pl.ds(k * SCAN_E, SCAN_E)],
                         st_dst.at[b], ssems.at[b])
        pltpu.async_copy(val_hbm.at[pl.ds(k * SCAN_E, SCAN_E)],
                         st_val.at[b], ssems.at[b])

    def wait_scan(b, k):
        pltpu.make_async_copy(src_hbm.at[pl.ds(k * SCAN_E, SCAN_E)],
                              st_src.at[b], ssems.at[b]).wait()
        pltpu.make_async_copy(dst_hbm.at[pl.ds(k * SCAN_E, SCAN_E)],
                              st_dst.at[b], ssems.at[b]).wait()
        pltpu.make_async_copy(val_hbm.at[pl.ds(k * SCAN_E, SCAN_E)],
                              st_val.at[b], ssems.at[b]).wait()

    def scan_chunk(b, bm1):
        @pl.loop(0, SCAN_E // 16, init_carry=bm1)
        def inner(g, bm1):
            sl = pl.ds(g * 16, 16)
            sv = st_src[b, sl]
            dv = st_dst[b, sl]
            vv = st_val[b, sl]
            owner = lax.shift_right_logical(dv * 6554, 22)
            m = owner == s
            pos = plsc.cumsum(m.astype(jnp.int32))
            idx = bm1 + pos
            mm = jnp.logical_and(m, idx < CAPG)
            plsc.store_scatter(bsrc, [idx], sv, mask=mm)
            plsc.store_scatter(bdst, [idx], dv - row0, mask=mm)
            plsc.store_scatter(bval, [idx], vv * (1.0 - ALPHA), mask=mm)
            pc = plsc.all_reduce_population_count(mm)
            return bm1 + pc

        return inner

    fire_scan(0, 0)
    fire_scan(1, 1)

    @pl.loop(0, NSCAN // 2, init_carry=jnp.full((16,), -1, jnp.int32))
    def scan_loop(i, bm1):
        k0 = 2 * i
        wait_scan(0, k0)
        bm1 = scan_chunk(0, bm1)

        @pl.when(k0 + 2 < NSCAN)
        def _():
            fire_scan(0, k0 + 2)

        wait_scan(1, k0 + 1)
        bm1 = scan_chunk(1, bm1)

        @pl.when(k0 + 3 < NSCAN)
        def _():
            fire_scan(1, k0 + 3)

        return bm1

    cnt = jnp.max(scan_loop) + 1

    zero16i = jnp.zeros((16,), jnp.int32)
    zero16f = jnp.zeros((16,), jnp.float32)

    @pl.loop(0, 512 // 16)
    def _(g):
        idx = cnt + g * 16 + iota
        mm = idx < CAP
        plsc.store_scatter(bsrc, [idx], zero16i, mask=mm)
        plsc.store_scatter(bdst, [idx], zero16i, mask=mm)
        plsc.store_scatter(bval, [idx], zero16f, mask=mm)

    nq = lax.shift_right_logical(cnt, 9) + 1

    pltpu.sync_copy(h_hbm.at[c, pl.ds(row0, ROWS_PT)],
                    zsp.at[pl.ds(row0, ROWS_PT)])
    plsc.subcore_barrier()

    nchunks = nq * NBUF

    def fire_gather(b, j):
        pltpu.async_copy(zsp.at[bsrc.at[pl.ds(j * CHUNK, CHUNK)]],
                         bufs.at[b], gsems.at[b])

    def do_chunk(b, j):
        pltpu.make_async_copy(zsp.at[bsrc.at[pl.ds(j * CHUNK, CHUNK)]],
                              bufs.at[b], gsems.at[b]).wait()

        @pl.loop(0, CHUNK // 16)
        def _(g):
            e0 = j * CHUNK + g * 16
            vv = bval[pl.ds(e0, 16)]
            dv = bdst[pl.ds(e0, 16)]
            for l in range(16):
                e = g * 16 + l
                vs = jnp.full((16,), vv[l], dtype=jnp.float32)
                rs = jnp.full((16,), dv[l], dtype=jnp.int32)
                a = bufs[b, e, pl.ds(0, 16)]
                bb = bufs[b, e, pl.ds(16, 16)]
                plsc.addupdate_scatter(acc, [rs, iota], a * vs)
                plsc.addupdate_scatter(acc, [rs, iota + 16], bb * vs)

        @pl.when(j + NBUF < nchunks)
        def _():
            fire_gather(b, j + NBUF)

    @pl.loop(0, K)
    def _(t):
        pltpu.sync_copy(h01_hbm.at[c, pl.ds(row0, ROWS_PT)], acc)

        for b in range(NBUF):
            fire_gather(b, b)

        @pl.loop(0, nq)
        def _(i):
            for b in range(NBUF):
                do_chunk(b, i * NBUF + b)

        plsc.subcore_barrier()
        pltpu.sync_copy(acc, zsp.at[pl.ds(row0, ROWS_PT)])
        plsc.subcore_barrier()

    pltpu.sync_copy(zsp.at[pl.ds(row0, ROWS_PT)],
                    out_hbm.at[c, pl.ds(row0, ROWS_PT)])


def _propagate(h2, h012, src_p, dst_p, val_p):
    mesh = plsc.VectorSubcoreMesh(core_axis_name="c", subcore_axis_name="s")
    cp = pltpu.CompilerParams(
        needs_layout_passes=False,
        use_tc_tiling_on_sc=False,
    )
    kfn = pl.kernel(
        _propagate_body,
        out_type=jax.ShapeDtypeStruct((NC, NP, HALF), jnp.float32),
        mesh=mesh,
        scratch_types=[
            pltpu.VMEM((CAP,), jnp.int32),
            pltpu.VMEM((CAP,), jnp.int32),
            pltpu.VMEM((CAP,), jnp.float32),
            pltpu.VMEM((ROWS_PT, HALF), jnp.float32),
            pltpu.VMEM((NBUF, CHUNK, HALF), jnp.float32),
            pltpu.VMEM((2, SCAN_E), jnp.int32),
            pltpu.VMEM((2, SCAN_E), jnp.int32),
            pltpu.VMEM((2, SCAN_E), jnp.float32),
            pltpu.VMEM_SHARED((NP, HALF), jnp.float32),
            pltpu.SemaphoreType.DMA((NBUF,)),
            pltpu.SemaphoreType.DMA((2,)),
        ],
        compiler_params=cp,
    )
    return kfn(h2, h012, src_p, dst_p, val_p)


def kernel(x, adj_indices, adj_values, W1, b1, W2, b2):
    h, h01 = _mlp(x, W1, b1, W2, b2)
    h2 = h.reshape(N, NC, HALF).transpose(1, 0, 2)
    h2 = jnp.pad(h2, ((0, 0), (0, NP - N), (0, 0)))
    h012 = h01.reshape(N, NC, HALF).transpose(1, 0, 2)
    h012 = jnp.pad(h012, ((0, 0), (0, NP - N), (0, 0)))

    pad = E_PAD - E
    src_p = jnp.concatenate([adj_indices[0], jnp.zeros((pad,), jnp.int32)])
    dst_p = jnp.concatenate([adj_indices[1],
                             jnp.full((pad,), NP, jnp.int32)])
    val_p = jnp.concatenate([adj_values, jnp.zeros((pad,), jnp.float32)])

    z2 = _propagate(h2, h012, src_p, dst_p, val_p)
    return z2[:, :N, :].transpose(1, 0, 2).reshape(N, D_OUT)


# speedup vs baseline: 2.2624x; 2.2624x over previous
import jax
import jax.numpy as jnp
from jax import lax
from jax.experimental import pallas as pl
from jax.experimental.pallas import tpu as pltpu
from jax.experimental.pallas import tpu_sc as plsc

N = 10000
E = 320000
D_IN = 128
HID = 64
D_OUT = 64
HALF = 32
ALPHA = 0.1
K = 10

NC = 2
NS = 16
CHUNK = 128
NCHUNK = 162
NBUF = 6
EPT = NCHUNK * CHUNK
NP = 10240
ROWS_PT = NP // NS

_BN = 2000


def _mlp_body(x_ref, w1_ref, b1_ref, w2_ref, b2_ref, o_ref, oa_ref):
    h = jnp.dot(x_ref[...], w1_ref[...], preferred_element_type=jnp.float32)
    h = jnp.maximum(h + b1_ref[...], 0.0)
    o = jnp.dot(h, w2_ref[...], preferred_element_type=jnp.float32)
    o = o + b2_ref[...]
    o_ref[...] = o
    oa_ref[...] = o * ALPHA


def _mlp(x, W1, b1, W2, b2):
    return pl.pallas_call(
        _mlp_body,
        grid=(N // _BN,),
        in_specs=[
            pl.BlockSpec((_BN, D_IN), lambda i: (i, 0)),
            pl.BlockSpec((D_IN, HID), lambda i: (0, 0)),
            pl.BlockSpec((1, HID), lambda i: (0, 0)),
            pl.BlockSpec((HID, D_OUT), lambda i: (0, 0)),
            pl.BlockSpec((1, D_OUT), lambda i: (0, 0)),
        ],
        out_specs=[pl.BlockSpec((_BN, D_OUT), lambda i: (i, 0)),
                   pl.BlockSpec((_BN, D_OUT), lambda i: (i, 0))],
        out_shape=[jax.ShapeDtypeStruct((N, D_OUT), jnp.float32),
                   jax.ShapeDtypeStruct((N, D_OUT), jnp.float32)],
    )(x, W1, b1.reshape(1, HID), W2, b2.reshape(1, D_OUT))


def _propagate_body(h_hbm, h01_hbm, src_hbm, dst_hbm, val_hbm, out_hbm,
                    src_v, dst_v, val_v, bufs, z0sp, z1sp,
                    gsems, ssems):
    c = lax.axis_index("c")
    s = lax.axis_index("s")

    pltpu.sync_copy(src_hbm.at[s], src_v)
    pltpu.sync_copy(dst_hbm.at[s], dst_v)
    pltpu.sync_copy(val_hbm.at[s], val_v)

    @pl.loop(0, NCHUNK)
    def _(j):
        @pl.loop(0, CHUNK // 16)
        def _(g):
            sl = (j, pl.ds(g * 16, 16))
            val_v[sl] = val_v[sl] * (1.0 - ALPHA)

    row0 = s * ROWS_PT
    pltpu.sync_copy(h_hbm.at[c, pl.ds(row0, ROWS_PT)],
                    z0sp.at[pl.ds(row0, ROWS_PT)])

    plsc.subcore_barrier()

    def scale_chunk(j, buf):
        @pl.loop(0, CHUNK // 16)
        def _(g):
            vv = val_v[j, pl.ds(g * 16, 16)]
            for l in range(16):
                e = g * 16 + l
                vs = jnp.full((16,), vv[l], dtype=jnp.float32)
                a = buf[e, pl.ds(0, 16)]
                buf[e, pl.ds(0, 16)] = a * vs
                b = buf[e, pl.ds(16, 16)]
                buf[e, pl.ds(16, 16)] = b * vs

    def one_step(cur, nxt):
        pltpu.sync_copy(h01_hbm.at[c, pl.ds(row0, ROWS_PT)],
                        nxt.at[pl.ds(row0, ROWS_PT)])
        plsc.subcore_barrier()

        for b in range(NBUF):
            pltpu.async_copy(cur.at[src_v.at[b]], bufs.at[b], gsems.at[b])

        @pl.loop(0, NCHUNK // NBUF)
        def _(i):
            for b in range(NBUF):
                j = i * NBUF + b
                pltpu.make_async_copy(
                    cur.at[src_v.at[j]], bufs.at[b], gsems.at[b]).wait()
                scale_chunk(j, bufs.at[b])
                pltpu.async_copy(
                    bufs.at[b], nxt.at[dst_v.at[j]], ssems.at[b], add=True)

                pb = (b - 1) % NBUF
                pj = j + NBUF - 1

                @pl.when(jnp.logical_and(j >= 1, pj < NCHUNK))
                def _():
                    pltpu.make_async_copy(
                        bufs.at[pb], nxt.at[dst_v.at[j - 1]],
                        ssems.at[pb]).wait()
                    pltpu.async_copy(
                        cur.at[src_v.at[pj]], bufs.at[pb], gsems.at[pb])

        for b in range(NBUF):
            jd = NCHUNK - NBUF + b
            pltpu.make_async_copy(
                bufs.at[b], nxt.at[dst_v.at[jd]], ssems.at[b]).wait()

        plsc.subcore_barrier()

    @pl.loop(0, K // 2)
    def _(t2):
        one_step(z0sp, z1sp)
        one_step(z1sp, z0sp)

    pltpu.sync_copy(z0sp.at[pl.ds(row0, ROWS_PT)],
                    out_hbm.at[c, pl.ds(row0, ROWS_PT)])


def _propagate(h2, h012, src3, dst3, val3):
    mesh = plsc.VectorSubcoreMesh(core_axis_name="c", subcore_axis_name="s")
    cp = pltpu.CompilerParams(
        needs_layout_passes=False,
        use_tc_tiling_on_sc=False,
    )
    kfn = pl.kernel(
        _propagate_body,
        out_type=jax.ShapeDtypeStruct((NC, NP, HALF), jnp.float32),
        mesh=mesh,
        scratch_types=[
            pltpu.VMEM((NCHUNK, CHUNK), jnp.int32),
            pltpu.VMEM((NCHUNK, CHUNK), jnp.int32),
            pltpu.VMEM((NCHUNK, CHUNK), jnp.float32),
            pltpu.VMEM((NBUF, CHUNK, HALF), jnp.float32),
            pltpu.VMEM_SHARED((NP, HALF), jnp.float32),
            pltpu.VMEM_SHARED((NP, HALF), jnp.float32),
            pltpu.SemaphoreType.DMA((NBUF,)),
            pltpu.SemaphoreType.DMA((NBUF,)),
        ],
        compiler_params=cp,
    )
    return kfn(h2, h012, src3, dst3, val3)


def kernel(x, adj_indices, adj_values, W1, b1, W2, b2):
    h, h01 = _mlp(x, W1, b1, W2, b2)
    h2 = h.reshape(N, NC, HALF).transpose(1, 0, 2)
    h2 = jnp.pad(h2, ((0, 0), (0, NP - N), (0, 0)))
    h012 = h01.reshape(N, NC, HALF).transpose(1, 0, 2)
    h012 = jnp.pad(h012, ((0, 0), (0, NP - N), (0, 0)))

    pad = NS * EPT - E
    src = jnp.concatenate([adj_indices[0], jnp.zeros((pad,), jnp.int32)])
    dst = jnp.concatenate([adj_indices[1], jnp.zeros((pad,), jnp.int32)])
    val = jnp.concatenate([adj_values, jnp.zeros((pad,), jnp.float32)])
    src3 = src.reshape(NS, NCHUNK, CHUNK)
    dst3 = dst.reshape(NS, NCHUNK, CHUNK)
    val3 = val.reshape(NS, NCHUNK, CHUNK)

    z2 = _propagate(h2, h012, src3, dst3, val3)
    return z2[:, :N, :].transpose(1, 0, 2).reshape(N, D_OUT)
